# baseline (speedup 1.0000x reference)
import functools

import jax
import jax.numpy as jnp
from jax import lax
from jax.experimental import pallas as pl
from jax.experimental.pallas import tpu as pltpu
from jax.experimental.pallas import tpu_sc as plsc

_NT = 16384
_S = 50
_D = 32
_V = 1000000
_NJ = (_V + 127) // 128
_VLAST = _V - 128 * (_NJ - 1)
_JMAIN = 7808
_JPW = _JMAIN // 32
_NB = 4
_NSTEP = _JPW // _NB

_INFO = plsc.get_sparse_core_info()
_NC = _INFO.num_cores
_NS = _INFO.num_subcores
_NW = _NC * _NS
_TPW = _NT // _NW
_NVEC = _TPW // 16


@functools.partial(
    pl.kernel,
    mesh=plsc.VectorSubcoreMesh(core_axis_name="c", subcore_axis_name="s"),
    out_type=jax.ShapeDtypeStruct((_V // 4, 128), jnp.float32),
    scratch_types=[
        pltpu.VMEM((_D, _NB * 128), jnp.float32),
        pltpu.VMEM((_NB * 32, 128), jnp.float32),
        pltpu.VMEM((_D, 128), jnp.float32),
        pltpu.VMEM((32, 128), jnp.float32),
    ],
    compiler_params=pltpu.CompilerParams(
        use_tc_tiling_on_sc=True, needs_layout_passes=False),
)
def _sc_transpose(wt_hbm, s_hbm, buf_v, so_v, buf1_v, so1_v):
    wid = lax.axis_index("s") * _NC + lax.axis_index("c")
    j0 = wid * _JPW
    iota = lax.iota(jnp.int32, 16)
    zero16 = iota * 0
    rows_even = iota
    rows_odd = iota + 16

    def body(i, _):
        jb = j0 + i * _NB
        cb = pl.multiple_of(jb * 128, 128)
        rb = pl.multiple_of(jb * 32, 32)
        pltpu.sync_copy(wt_hbm.at[:, pl.ds(cb, _NB * 128)], buf_v)

        def rbody(r, _):
            for k in range(8):
                rows = rows_even if k % 2 == 0 else rows_odd
                col = zero16 + (4 * r + k // 2)
                so_v[r, pl.ds(16 * k, 16)] = plsc.load_gather(
                    buf_v, [rows, col])
            return ()

        lax.fori_loop(0, _NB * 32, rbody, (), unroll=False)
        pltpu.sync_copy(so_v, s_hbm.at[pl.ds(rb, _NB * 32)])
        return ()

    lax.fori_loop(0, _NSTEP, body, (), unroll=False)

    @pl.when(wid < _NJ - _JMAIN - 1)
    def _tail_full():
        j = _JMAIN + wid
        pltpu.sync_copy(wt_hbm.at[:, pl.ds(j * 128, 128)], buf1_v)
        for r in range(32):
            for k in range(8):
                rows = rows_even if k % 2 == 0 else rows_odd
                col = zero16 + (4 * r + k // 2)
                so1_v[r, pl.ds(16 * k, 16)] = plsc.load_gather(
                    buf1_v, [rows, col])
        pltpu.sync_copy(so1_v, s_hbm.at[pl.ds(j * 32, 32)])


@functools.partial(
    pl.kernel,
    mesh=plsc.VectorSubcoreMesh(core_axis_name="c", subcore_axis_name="s"),
    out_type=jax.ShapeDtypeStruct((_S, _D, _NT), jnp.float32),
    scratch_types=[
        pltpu.VMEM((_TPW * _S,), jnp.int32),
        pltpu.VMEM((_TPW,), jnp.int32),
        pltpu.VMEM((_TPW,), jnp.int32),
        pltpu.VMEM((_TPW, 128), jnp.float32),
        pltpu.VMEM((_D, _TPW), jnp.float32),
        pltpu.SemaphoreType.DMA,
    ],
    compiler_params=pltpu.CompilerParams(
        use_tc_tiling_on_sc=True, needs_layout_passes=False),
)
def _sc_embed(idx_hbm, w128_hbm, out_hbm, ids_v, idg_v, idm_v, g_v, o_v, sem):
    wid = lax.axis_index("s") * _NC + lax.axis_index("c")
    t0 = wid * _TPW
    pltpu.sync_copy(idx_hbm.at[pl.ds(t0 * _S, _TPW * _S)], ids_v)

    iota = lax.iota(jnp.int32, 16)
    stride_s = iota * _S

    def body(s, _):
        for k in range(_NVEC):
            idxvec = stride_s + (16 * k * _S + s)
            ids = plsc.load_gather(ids_v, [idxvec])
            idg_v[pl.ds(16 * k, 16)] = ids >> 2
            idm_v[pl.ds(16 * k, 16)] = (ids & 3) << 5
        pltpu.async_copy(w128_hbm.at[idg_v], g_v, sem).wait()
        for k in range(_NVEC):
            rows = iota + (16 * k)
            cols0 = idm_v[pl.ds(16 * k, 16)]
            for d in range(_D):
                o_v[d, pl.ds(16 * k, 16)] = plsc.load_gather(
                    g_v, [rows, cols0 + d])
        pltpu.sync_copy(o_v, out_hbm.at[s, :, pl.ds(t0, _TPW)])
        return ()

    lax.fori_loop(0, _S, body, (), unroll=False)


def kernel(token_ids, weight):
    flat = token_ids.reshape(-1).astype(jnp.int32)
    w128 = _sc_transpose(weight.T)
    tail = weight[(_NJ - 1) * 128:].reshape(_VLAST // 4, 128)
    w128 = lax.dynamic_update_slice(w128, tail, ((_NJ - 1) * 32, 0))
    out = _sc_embed(flat, w128)
    return jnp.transpose(out, (2, 0, 1))


# speedup vs baseline: 1.5165x; 1.5165x over previous
import functools

import jax
import jax.numpy as jnp
from jax import lax
from jax.experimental import pallas as pl
from jax.experimental.pallas import tpu as pltpu
from jax.experimental.pallas import tpu_sc as plsc

_NT = 16384
_S = 50
_D = 32
_V = 1000000

_INFO = plsc.get_sparse_core_info()
_NC = _INFO.num_cores
_NS = _INFO.num_subcores
_NW = _NC * _NS
_TPW = _NT // _NW
_HC = _TPW // 2
_KC = _HC // 16


@functools.partial(
    pl.kernel,
    mesh=plsc.VectorSubcoreMesh(core_axis_name="c", subcore_axis_name="s"),
    out_type=jax.ShapeDtypeStruct((_S, _D, _NT), jnp.float32),
    scratch_types=[
        pltpu.VMEM((_TPW * _S,), jnp.int32),
        pltpu.VMEM((_HC,), jnp.int32),
        pltpu.VMEM((_HC,), jnp.int32),
        pltpu.VMEM((_HC,), jnp.int32),
        pltpu.VMEM((_HC,), jnp.int32),
        pltpu.VMEM((_HC, 128), jnp.float32),
        pltpu.VMEM((_HC, 128), jnp.float32),
        pltpu.VMEM((_D, _HC), jnp.float32),
        pltpu.VMEM((_D, _HC), jnp.float32),
        pltpu.SemaphoreType.DMA,
        pltpu.SemaphoreType.DMA,
        pltpu.SemaphoreType.DMA,
        pltpu.SemaphoreType.DMA,
    ],
    compiler_params=pltpu.CompilerParams(
        use_tc_tiling_on_sc=True, needs_layout_passes=False),
)
def _sc_embed(idx_hbm, w128_hbm, out_hbm, ids_v, idga_v, idma_v, idgb_v,
              idmb_v, ga_v, gb_v, oa_v, ob_v, gsa, gsb, ssa, ssb):
    wid = lax.axis_index("s") * _NC + lax.axis_index("c")
    t0 = pl.multiple_of(wid * _TPW, _TPW)
    pltpu.sync_copy(idx_hbm.at[pl.ds(t0 * _S, _TPW * _S)], ids_v)

    iota = lax.iota(jnp.int32, 16)
    stride_s = iota * _S

    def build(s, h, idg_v, idm_v):
        for k in range(_KC):
            idxvec = stride_s + ((_HC * h + 16 * k) * _S + s)
            ids = plsc.load_gather(ids_v, [idxvec])
            idg_v[pl.ds(16 * k, 16)] = ids >> 2
            idm_v[pl.ds(16 * k, 16)] = (ids & 3) << 5

    def extract(g_v, idm_v, o_v):
        def kbody(k, _):
            rows = iota + 16 * k
            kk = pl.multiple_of(16 * k, 16)
            cols0 = idm_v[pl.ds(kk, 16)]
            for d in range(_D):
                o_v[d, pl.ds(kk, 16)] = plsc.load_gather(
                    g_v, [rows, cols0 + d])
            return ()
        lax.fori_loop(0, _KC, kbody, (), unroll=False)

    build(0, 0, idga_v, idma_v)
    pltpu.async_copy(w128_hbm.at[idga_v], ga_v, gsa)

    def body(s, _):
        build(s, 1, idgb_v, idmb_v)
        pltpu.async_copy(w128_hbm.at[idgb_v], gb_v, gsb)
        pltpu.make_async_copy(w128_hbm.at[idga_v], ga_v, gsa).wait()

        @pl.when(s > 0)
        def _():
            pltpu.make_async_copy(
                oa_v, out_hbm.at[0, :, pl.ds(t0, _HC)], ssa).wait()
        extract(ga_v, idma_v, oa_v)
        pltpu.async_copy(oa_v, out_hbm.at[s, :, pl.ds(t0, _HC)], ssa)

        @pl.when(s < _S - 1)
        def _():
            build(s + 1, 0, idga_v, idma_v)
            pltpu.async_copy(w128_hbm.at[idga_v], ga_v, gsa)

        pltpu.make_async_copy(w128_hbm.at[idgb_v], gb_v, gsb).wait()

        @pl.when(s > 0)
        def _():
            pltpu.make_async_copy(
                ob_v, out_hbm.at[0, :, pl.ds(t0 + _HC, _HC)], ssb).wait()
        extract(gb_v, idmb_v, ob_v)
        pltpu.async_copy(ob_v, out_hbm.at[s, :, pl.ds(t0 + _HC, _HC)], ssb)
        return ()

    lax.fori_loop(0, _S, body, (), unroll=False)
    pltpu.make_async_copy(oa_v, out_hbm.at[0, :, pl.ds(t0, _HC)], ssa).wait()
    pltpu.make_async_copy(
        ob_v, out_hbm.at[0, :, pl.ds(t0 + _HC, _HC)], ssb).wait()


def kernel(token_ids, weight):
    flat = token_ids.reshape(-1).astype(jnp.int32)
    w128 = weight.reshape(_V // 4, 128)
    out = _sc_embed(flat, w128)
    return jnp.transpose(out, (2, 0, 1))
